# baseline (speedup 1.0000x reference)
import functools

import jax
import jax.numpy as jnp
from jax import lax
from jax.experimental import pallas as pl
from jax.experimental.pallas import tpu as pltpu
from jax.experimental.pallas import tpu_sc as plsc

B = 4096
F = 26
D = 64
NC = 2
NS = 16
NW = NC * NS
SPT = B // NW
G = 4
NG = SPT // G
L = 16


def _sc_body(idx_s_hbm, emb_hbm, lin_hbm,
             s_out_hbm, q_out_hbm, l_out_hbm,
             idx_s_v, ebuf0, ebuf1, lbuf, s_v, q_v, l_v,
             sem0, sem1, lsem):
    wid = lax.axis_index("s") * NC + lax.axis_index("c")
    base = wid * SPT

    pltpu.sync_copy(idx_s_hbm.at[wid], idx_s_v)

    for g in range(NG):
        pltpu.make_async_copy(
            lin_hbm.at[idx_s_v.at[g]],
            lbuf.at[pl.ds(g * G * F, G * F)], lsem).start()

    pltpu.make_async_copy(emb_hbm.at[idx_s_v.at[0]], ebuf0, sem0).start()
    pltpu.make_async_copy(emb_hbm.at[idx_s_v.at[1]], ebuf1, sem1).start()

    def accumulate(g, ebuf):
        for k in range(G):
            row0 = k * F
            orow = g * G + k
            for c in range(D // L):
                sl = pl.ds(c * L, L)
                s = ebuf[row0, sl]
                q = s * s
                for j in range(1, F):
                    r = ebuf[row0 + j, sl]
                    s = s + r
                    q = q + r * r
                s_v[orow, sl] = s
                q_v[orow, sl] = q

    def step(t, carry):
        for b, (ebuf, sem) in enumerate(((ebuf0, sem0), (ebuf1, sem1))):
            g = 2 * t + b
            pltpu.make_async_copy(emb_hbm.at[idx_s_v.at[g]], ebuf, sem).wait()
            accumulate(g, ebuf)

            @pl.when(t < NG // 2 - 1)
            def _():
                pltpu.make_async_copy(
                    emb_hbm.at[idx_s_v.at[g + 2]], ebuf, sem).start()
        return carry

    lax.fori_loop(0, NG // 2, step, 0)

    for g in range(NG):
        pltpu.make_async_copy(
            lin_hbm.at[idx_s_v.at[g]],
            lbuf.at[pl.ds(g * G * F, G * F)], lsem).wait()
    for cc in range(SPT // L):
        sl = pl.ds(cc * L, L)
        bidx = (cc * L + lax.iota(jnp.int32, L)) * F
        a = plsc.load_gather(lbuf, [bidx])
        for j in range(1, F):
            a = a + plsc.load_gather(lbuf, [bidx + j])
        l_v[sl] = a

    pltpu.sync_copy(s_v, s_out_hbm.at[pl.ds(base, SPT)])
    pltpu.sync_copy(q_v, q_out_hbm.at[pl.ds(base, SPT)])
    pltpu.sync_copy(l_v, l_out_hbm.at[pl.ds(base, SPT)])


_sc_bag = functools.partial(
    pl.kernel,
    out_type=(
        jax.ShapeDtypeStruct((B, D), jnp.float32),
        jax.ShapeDtypeStruct((B, D), jnp.float32),
        jax.ShapeDtypeStruct((B,), jnp.float32),
    ),
    mesh=plsc.VectorSubcoreMesh(
        core_axis_name="c", subcore_axis_name="s",
        num_cores=NC, num_subcores=NS),
    compiler_params=pltpu.CompilerParams(
        use_tc_tiling_on_sc=False, needs_layout_passes=False),
    scratch_types=[
        pltpu.VMEM((NG, G * F), jnp.int32),
        pltpu.VMEM((G * F, D), jnp.float32),
        pltpu.VMEM((G * F, D), jnp.float32),
        pltpu.VMEM((SPT * F,), jnp.float32),
        pltpu.VMEM((SPT, D), jnp.float32),
        pltpu.VMEM((SPT, D), jnp.float32),
        pltpu.VMEM((SPT,), jnp.float32),
        pltpu.SemaphoreType.DMA,
        pltpu.SemaphoreType.DMA,
        pltpu.SemaphoreType.DMA,
    ],
)(_sc_body)


def _tc_body(s_ref, q_ref, l_ref, num_ref, w1_ref, nlw_ref, fmw_ref, bias_ref,
             o_ref):
    num = num_ref[:]
    w1 = w1_ref[:]
    s = s_ref[:] + jnp.dot(num, w1, preferred_element_type=jnp.float32)
    q = q_ref[:] + jnp.dot(num * num, w1 * w1,
                           preferred_element_type=jnp.float32)
    fm = 0.5 * (s * s - q)
    o_ref[:] = (jnp.dot(fm, fmw_ref[:], preferred_element_type=jnp.float32)
                + jnp.dot(num, nlw_ref[:], preferred_element_type=jnp.float32)
                + l_ref[:] + bias_ref[:])


def kernel(categorical, numerical, num_lin_W, num_lin_b, cat_lin_table,
           cat_lin_bias, num_emb_W, cat_emb_table, fm_W, fm_b):
    offsets = jnp.arange(F, dtype=jnp.int32) * 100000
    idx = categorical.astype(jnp.int32) + offsets[None, :]
    idx_smaj = idx.reshape(NW, NG, G * F)

    s_sum, q_sum, l_sum = _sc_bag(
        idx_smaj, cat_emb_table, cat_lin_table.reshape(-1))

    bias = (num_lin_b + cat_lin_bias + fm_b).reshape(1, 1)
    out = pl.pallas_call(
        _tc_body,
        out_shape=jax.ShapeDtypeStruct((B, 1), jnp.float32),
    )(s_sum, q_sum, l_sum.reshape(B, 1), numerical,
      num_emb_W.reshape(-1, D), num_lin_W, fm_W, bias)
    return out


# speedup vs baseline: 1.0003x; 1.0003x over previous
import functools

import jax
import jax.numpy as jnp
from jax import lax
from jax.experimental import pallas as pl
from jax.experimental.pallas import tpu as pltpu
from jax.experimental.pallas import tpu_sc as plsc

B = 4096
F = 26
D = 64
NC = 2
NS = 16
NW = NC * NS
SPT = B // NW
G = 4
NG = SPT // G
L = 16


def _sc_body(cat_t_hbm, emb_hbm, lin_hbm,
             s_out_hbm, q_out_hbm, l_out_hbm,
             cat_v, idx_s_v, ebuf0, ebuf1, lbuf, s_v, q_v, l_v,
             sem0, sem1, lsem):
    wid = lax.axis_index("s") * NC + lax.axis_index("c")
    base = wid * SPT

    pltpu.sync_copy(cat_t_hbm.at[:, pl.ds(base, SPT)], cat_v)
    ii = lax.iota(jnp.int32, L)
    g_hi = ii >> 2
    w_lo = (ii & 3) * F
    for cc in range(SPT // L):
        g_vec = g_hi + (cc * L // G)
        for j in range(F):
            v = cat_v[j, pl.ds(cc * L, L)] + (j * 100000)
            plsc.store_scatter(idx_s_v, [g_vec, w_lo + j], v)

    for g in range(NG):
        pltpu.make_async_copy(
            lin_hbm.at[idx_s_v.at[g]],
            lbuf.at[pl.ds(g * G * F, G * F)], lsem).start()

    pltpu.make_async_copy(emb_hbm.at[idx_s_v.at[0]], ebuf0, sem0).start()
    pltpu.make_async_copy(emb_hbm.at[idx_s_v.at[1]], ebuf1, sem1).start()

    def accumulate(g, ebuf):
        for k in range(G):
            row0 = k * F
            orow = g * G + k
            for c in range(D // L):
                sl = pl.ds(c * L, L)
                s = ebuf[row0, sl]
                q = s * s
                for j in range(1, F):
                    r = ebuf[row0 + j, sl]
                    s = s + r
                    q = q + r * r
                s_v[orow, sl] = s
                q_v[orow, sl] = q

    def step(t, carry):
        for b, (ebuf, sem) in enumerate(((ebuf0, sem0), (ebuf1, sem1))):
            g = 2 * t + b
            pltpu.make_async_copy(emb_hbm.at[idx_s_v.at[g]], ebuf, sem).wait()
            accumulate(g, ebuf)

            @pl.when(t < NG // 2 - 1)
            def _():
                pltpu.make_async_copy(
                    emb_hbm.at[idx_s_v.at[g + 2]], ebuf, sem).start()
        return carry

    lax.fori_loop(0, NG // 2, step, 0)

    for g in range(NG):
        pltpu.make_async_copy(
            lin_hbm.at[idx_s_v.at[g]],
            lbuf.at[pl.ds(g * G * F, G * F)], lsem).wait()
    for cc in range(SPT // L):
        sl = pl.ds(cc * L, L)
        bidx = (cc * L + lax.iota(jnp.int32, L)) * F
        a = plsc.load_gather(lbuf, [bidx])
        for j in range(1, F):
            a = a + plsc.load_gather(lbuf, [bidx + j])
        l_v[sl] = a

    pltpu.sync_copy(s_v, s_out_hbm.at[pl.ds(base, SPT)])
    pltpu.sync_copy(q_v, q_out_hbm.at[pl.ds(base, SPT)])
    pltpu.sync_copy(l_v, l_out_hbm.at[pl.ds(base, SPT)])


_sc_bag = functools.partial(
    pl.kernel,
    out_type=(
        jax.ShapeDtypeStruct((B, D), jnp.float32),
        jax.ShapeDtypeStruct((B, D), jnp.float32),
        jax.ShapeDtypeStruct((B,), jnp.float32),
    ),
    mesh=plsc.VectorSubcoreMesh(
        core_axis_name="c", subcore_axis_name="s",
        num_cores=NC, num_subcores=NS),
    compiler_params=pltpu.CompilerParams(
        use_tc_tiling_on_sc=False, needs_layout_passes=False),
    scratch_types=[
        pltpu.VMEM((F, SPT), jnp.int32),
        pltpu.VMEM((NG, G * F), jnp.int32),
        pltpu.VMEM((G * F, D), jnp.float32),
        pltpu.VMEM((G * F, D), jnp.float32),
        pltpu.VMEM((SPT * F,), jnp.float32),
        pltpu.VMEM((SPT, D), jnp.float32),
        pltpu.VMEM((SPT, D), jnp.float32),
        pltpu.VMEM((SPT,), jnp.float32),
        pltpu.SemaphoreType.DMA,
        pltpu.SemaphoreType.DMA,
        pltpu.SemaphoreType.DMA,
    ],
)(_sc_body)


def _tc_body(s_ref, q_ref, l_ref, num_ref, w1_ref, nlw_ref, fmw_ref, bias_ref,
             o_ref):
    num = num_ref[:]
    w1 = w1_ref[:]
    s = s_ref[:] + jnp.dot(num, w1, preferred_element_type=jnp.float32)
    q = q_ref[:] + jnp.dot(num * num, w1 * w1,
                           preferred_element_type=jnp.float32)
    fm = 0.5 * (s * s - q)
    o_ref[:] = (jnp.dot(fm, fmw_ref[:], preferred_element_type=jnp.float32)
                + jnp.dot(num, nlw_ref[:], preferred_element_type=jnp.float32)
                + l_ref[:] + bias_ref[:])


def kernel(categorical, numerical, num_lin_W, num_lin_b, cat_lin_table,
           cat_lin_bias, num_emb_W, cat_emb_table, fm_W, fm_b):
    s_sum, q_sum, l_sum = _sc_bag(
        categorical.T, cat_emb_table, cat_lin_table.T.reshape(-1))

    bias = (num_lin_b + cat_lin_bias + fm_b).reshape(1, 1)
    out = pl.pallas_call(
        _tc_body,
        out_shape=jax.ShapeDtypeStruct((B, 1), jnp.float32),
    )(s_sum, q_sum, l_sum.reshape(B, 1), numerical,
      num_emb_W.reshape(-1, D), num_lin_W, fm_W, bias)
    return out
